# baseline (speedup 1.0000x reference)
import functools

import jax
import jax.numpy as jnp
from jax import lax
from jax.experimental import pallas as pl
from jax.experimental.pallas import tpu as pltpu
from jax.experimental.pallas import tpu_sc as plsc

T = 2048
H = 768
F = 3072
E = 8
TOPK = 2
BM = 128
NBLK = 40
NPAD = NBLK * BM

NTILES = 32
TOK_PT = T // NTILES
SLOT_PT = TOK_PT * TOPK
LANES = 16


def _router_body(x_ref, wr_ref, route_ref, hist_ref):
    logits = jnp.dot(x_ref[...], wr_ref[...], preferred_element_type=jnp.float32)
    lane = lax.broadcasted_iota(jnp.int32, logits.shape, 1)
    neg = jnp.float32(-jnp.inf)
    logits = jnp.where(lane < E, logits, neg)
    m1 = jnp.max(logits, axis=1, keepdims=True)
    i1 = jnp.min(jnp.where(logits == m1, lane, 2 * E), axis=1, keepdims=True)
    l2 = jnp.where(lane == i1, neg, logits)
    m2 = jnp.max(l2, axis=1, keepdims=True)
    i2 = jnp.min(jnp.where(l2 == m2, lane, 2 * E), axis=1, keepdims=True)
    w1 = jax.nn.sigmoid(m1 - m2)
    w2 = jax.nn.sigmoid(m2 - m1)
    route_ref[...] = jnp.where(
        lane == 0, i1.astype(jnp.float32),
        jnp.where(lane == 1, i2.astype(jnp.float32),
                  jnp.where(lane == 2, w1, w2)))
    onehot = (lane == i1).astype(jnp.float32) + (lane == i2).astype(jnp.float32)
    hist_ref[0] = jnp.sum(onehot, axis=0, keepdims=True)


def _router(x, Wr):
    wr_pad = jnp.zeros((H, 128), jnp.float32).at[:, :E].set(Wr)
    return pl.pallas_call(
        _router_body,
        grid=(NTILES,),
        in_specs=[
            pl.BlockSpec((TOK_PT, H), lambda g: (g, 0)),
            pl.BlockSpec((H, 128), lambda g: (0, 0)),
        ],
        out_specs=[
            pl.BlockSpec((TOK_PT, 128), lambda g: (g, 0)),
            pl.BlockSpec((1, 1, 128), lambda g: (g, 0, 0)),
        ],
        out_shape=[
            jax.ShapeDtypeStruct((T, 128), jnp.float32),
            jax.ShapeDtypeStruct((NTILES, 1, 128), jnp.float32),
        ],
    )(x, wr_pad)


def _slot_experts(rbuf, j):
    lanes = lax.broadcasted_iota(jnp.int32, (LANES,), 0)
    slot = 16 * j + lanes
    tok = slot // TOPK
    col = slot % TOPK
    return plsc.load_gather(rbuf, [tok * 128 + col]).astype(jnp.int32)


def _dispatch_kernel(route_hbm, hist_hbm, x_hbm, xpad_hbm, pos_hbm, cnt_hbm,
                     wpad_hbm,
                     rbuf, histbuf, table, posbuf, idx_a, idx_b, xrows, cntbuf,
                     wbuf, sem):
    wid = lax.axis_index("s") * 2 + lax.axis_index("c")
    lanes = lax.broadcasted_iota(jnp.int32, (LANES,), 0)
    zero = jnp.zeros((LANES,), jnp.int32)

    pltpu.sync_copy(route_hbm.at[pl.ds(wid * TOK_PT * 128, TOK_PT * 128)], rbuf)
    pltpu.sync_copy(hist_hbm, histbuf)

    totals = zero
    mybase = zero
    widv = jnp.full((LANES,), wid, jnp.int32)
    for wp in range(NTILES):
        row = histbuf[wp, pl.ds(0, LANES)].astype(jnp.int32)
        mybase = mybase + jnp.where(jnp.full((LANES,), wp, jnp.int32) < widv, row, zero)
        totals = totals + row
    tiles_e = (totals + (BM - 1)) // BM
    pad_off = BM * (plsc.cumsum(tiles_e) - tiles_e)
    table[...] = pad_off + mybase

    @pl.when(wid == 0)
    def _():
        cntbuf[...] = totals
        pltpu.sync_copy(cntbuf, cnt_hbm)

    carry = [jnp.int32(0)] * E
    for j in range(SLOT_PT // LANES):
        e_vec = _slot_experts(rbuf, j)
        slot = 16 * j + lanes
        tok = slot // TOPK
        wv = plsc.load_gather(rbuf, [tok * 128 + 2 + (slot % TOPK)])
        wbuf[pl.ds(16 * j, LANES)] = wv
        posv = zero
        for e in range(E):
            m = (e_vec == e)
            mi = m.astype(jnp.int32)
            incl = plsc.cumsum(mi)
            posv = jnp.where(m, carry[e] + (incl - mi), posv)
            carry[e] = carry[e] + jnp.sum(mi)
        base = plsc.load_gather(table, [e_vec])
        gpos = base + posv
        posbuf[pl.ds(16 * j, LANES)] = gpos
        evenm = (lanes % 2) == 0
        dest = (16 * j + lanes) // 2
        plsc.store_scatter(idx_a, [dest], gpos, mask=evenm)
        plsc.store_scatter(idx_b, [dest], gpos, mask=~evenm)

    pltpu.sync_copy(posbuf, pos_hbm.at[pl.ds(wid * SLOT_PT, SLOT_PT)])

    pltpu.sync_copy(x_hbm.at[pl.ds(wid * TOK_PT, TOK_PT)], xrows)
    pltpu.async_copy(xrows, xpad_hbm.at[idx_a], sem).wait()
    pltpu.async_copy(xrows, xpad_hbm.at[idx_b], sem).wait()
    pltpu.async_copy(wbuf, wpad_hbm.at[posbuf], sem).wait()


def _dispatch(route, hist, x):
    mesh = plsc.VectorSubcoreMesh(core_axis_name="c", subcore_axis_name="s")
    f = functools.partial(
        pl.kernel,
        mesh=mesh,
        compiler_params=pltpu.CompilerParams(needs_layout_passes=False),
        out_type=[
            jax.ShapeDtypeStruct((NPAD, H), jnp.float32),
            jax.ShapeDtypeStruct((T * TOPK,), jnp.int32),
            jax.ShapeDtypeStruct((LANES,), jnp.int32),
            jax.ShapeDtypeStruct((NPAD,), jnp.float32),
        ],
        scratch_types=[
            pltpu.VMEM((TOK_PT * 128,), jnp.float32),
            pltpu.VMEM((NTILES, 128), jnp.float32),
            pltpu.VMEM((LANES,), jnp.int32),
            pltpu.VMEM((SLOT_PT,), jnp.int32),
            pltpu.VMEM((TOK_PT,), jnp.int32),
            pltpu.VMEM((TOK_PT,), jnp.int32),
            pltpu.VMEM((TOK_PT, H), jnp.float32),
            pltpu.VMEM((LANES,), jnp.int32),
            pltpu.VMEM((SLOT_PT,), jnp.float32),
            pltpu.SemaphoreType.DMA,
        ],
    )
    return f(_dispatch_kernel)(route, hist, x)


def _ffn_body(bexp_ref, act_ref, x_ref, w1_ref, b1_ref, w2_ref, b2_ref, w_ref,
              o_ref):
    @pl.when(act_ref[pl.program_id(0)] != 0)
    def _():
        xb = x_ref[...].astype(jnp.bfloat16)
        h = jnp.dot(xb, w1_ref[0], preferred_element_type=jnp.float32)
        h = jax.nn.gelu(h + b1_ref[0]).astype(jnp.bfloat16)
        y = jnp.dot(h, w2_ref[0], preferred_element_type=jnp.float32)
        o_ref[...] = (y + b2_ref[0]) * w_ref[...]


def _grouped_ffn(x_pad, W1, b1, W2, b2, w_pad, bexp, act):
    grid_spec = pltpu.PrefetchScalarGridSpec(
        num_scalar_prefetch=2,
        grid=(NBLK,),
        in_specs=[
            pl.BlockSpec((BM, H), lambda g, be, ac: (g, 0)),
            pl.BlockSpec((1, H, F), lambda g, be, ac: (be[g], 0, 0)),
            pl.BlockSpec((1, 1, F), lambda g, be, ac: (be[g], 0, 0)),
            pl.BlockSpec((1, F, H), lambda g, be, ac: (be[g], 0, 0)),
            pl.BlockSpec((1, 1, H), lambda g, be, ac: (be[g], 0, 0)),
            pl.BlockSpec((BM, 1), lambda g, be, ac: (g, 0)),
        ],
        out_specs=pl.BlockSpec((BM, H), lambda g, be, ac: (g, 0)),
    )
    return pl.pallas_call(
        _ffn_body,
        grid_spec=grid_spec,
        out_shape=jax.ShapeDtypeStruct((NPAD, H), jnp.float32),
    )(bexp, act, x_pad, W1, b1.reshape(E, 1, F), W2, b2.reshape(E, 1, H),
      w_pad.reshape(NPAD, 1))


CHUNK = 32


def _combine_kernel(ys_hbm, pos_hbm, out_hbm, idx, rows, outb, sem):
    wid = lax.axis_index("s") * 2 + lax.axis_index("c")
    lanes = lax.broadcasted_iota(jnp.int32, (LANES,), 0)

    for c in range(TOK_PT // CHUNK):
        base_tok = wid * TOK_PT + c * CHUNK
        pltpu.sync_copy(pos_hbm.at[pl.ds(2 * base_tok, 2 * CHUNK)], idx)
        pltpu.async_copy(ys_hbm.at[idx], rows, sem).wait()

        def body(i, _):
            for j in range(H // LANES):
                a = rows[2 * i, pl.ds(j * LANES, LANES)]
                b = rows[2 * i + 1, pl.ds(j * LANES, LANES)]
                outb[i, pl.ds(j * LANES, LANES)] = a + b
            return 0

        lax.fori_loop(0, CHUNK, body, 0)
        pltpu.sync_copy(outb, out_hbm.at[pl.ds(base_tok, CHUNK)])


def _combine(ys_pad, pos):
    mesh = plsc.VectorSubcoreMesh(core_axis_name="c", subcore_axis_name="s")
    f = functools.partial(
        pl.kernel,
        mesh=mesh,
        compiler_params=pltpu.CompilerParams(needs_layout_passes=False),
        out_type=jax.ShapeDtypeStruct((T, H), jnp.float32),
        scratch_types=[
            pltpu.VMEM((2 * CHUNK,), jnp.int32),
            pltpu.VMEM((2 * CHUNK, H), jnp.float32),
            pltpu.VMEM((CHUNK, H), jnp.float32),
            pltpu.SemaphoreType.DMA,
        ],
    )
    return f(_combine_kernel)(ys_pad, pos)


def kernel(x, Wr, W1, b1, W2, b2):
    route, hist = _router(x, Wr)
    x_pad, pos, counts, w_pad = _dispatch(
        route.reshape(-1), hist.reshape(NTILES, 128), x)

    counts8 = counts[:E]
    tiles = (counts8 + BM - 1) // BM
    cum_tiles = jnp.cumsum(tiles)
    g_range = jnp.arange(NBLK, dtype=jnp.int32)
    bexp = jnp.minimum(
        jnp.searchsorted(cum_tiles, g_range, side="right"), E - 1
    ).astype(jnp.int32)
    act = (g_range < cum_tiles[-1]).astype(jnp.int32)

    ys_pad = _grouped_ffn(x_pad, W1.astype(jnp.bfloat16), b1,
                          W2.astype(jnp.bfloat16), b2, w_pad, bexp, act)
    return _combine(ys_pad, pos)


# speedup vs baseline: 1.1178x; 1.1178x over previous
import functools

import jax
import jax.numpy as jnp
from jax import lax
from jax.experimental import pallas as pl
from jax.experimental.pallas import tpu as pltpu
from jax.experimental.pallas import tpu_sc as plsc

T = 2048
H = 768
F = 3072
E = 8
TOPK = 2
BM = 128
NBLK = 40
NPAD = NBLK * BM

NTILES = 32
TOK_PT = T // NTILES
SLOT_PT = TOK_PT * TOPK
LANES = 16


def _router_body(x_ref, wr_ref, route_ref, hist_ref):
    logits = jnp.dot(x_ref[...], wr_ref[...], preferred_element_type=jnp.float32)
    lane = lax.broadcasted_iota(jnp.int32, logits.shape, 1)
    neg = jnp.float32(-jnp.inf)
    logits = jnp.where(lane < E, logits, neg)
    m1 = jnp.max(logits, axis=1, keepdims=True)
    i1 = jnp.min(jnp.where(logits == m1, lane, 2 * E), axis=1, keepdims=True)
    l2 = jnp.where(lane == i1, neg, logits)
    m2 = jnp.max(l2, axis=1, keepdims=True)
    i2 = jnp.min(jnp.where(l2 == m2, lane, 2 * E), axis=1, keepdims=True)
    w1 = jax.nn.sigmoid(m1 - m2)
    w2 = jax.nn.sigmoid(m2 - m1)
    route_ref[...] = jnp.where(
        lane == 0, i1.astype(jnp.float32),
        jnp.where(lane == 1, i2.astype(jnp.float32),
                  jnp.where(lane == 2, w1, w2)))
    onehot = (lane == i1).astype(jnp.float32) + (lane == i2).astype(jnp.float32)
    hist_ref[0] = jnp.sum(onehot, axis=0, keepdims=True)


def _router(x, Wr):
    wr_pad = jnp.zeros((H, 128), jnp.float32).at[:, :E].set(Wr)
    return pl.pallas_call(
        _router_body,
        grid=(NTILES,),
        in_specs=[
            pl.BlockSpec((TOK_PT, H), lambda g: (g, 0)),
            pl.BlockSpec((H, 128), lambda g: (0, 0)),
        ],
        out_specs=[
            pl.BlockSpec((TOK_PT, 128), lambda g: (g, 0)),
            pl.BlockSpec((1, 1, 128), lambda g: (g, 0, 0)),
        ],
        out_shape=[
            jax.ShapeDtypeStruct((T, 128), jnp.float32),
            jax.ShapeDtypeStruct((NTILES, 1, 128), jnp.float32),
        ],
    )(x, wr_pad)


def _slot_experts(rbuf, j):
    lanes = lax.broadcasted_iota(jnp.int32, (LANES,), 0)
    slot = 16 * j + lanes
    tok = slot // TOPK
    col = slot % TOPK
    return plsc.load_gather(rbuf, [tok * 128 + col]).astype(jnp.int32)


def _dispatch_kernel(route_hbm, hist_hbm, x_hbm, xpad_hbm, pos_hbm, cnt_hbm,
                     wpad_hbm,
                     rbuf, histbuf, table, posbuf, idx_a, idx_b, xrows, cntbuf,
                     wbuf, sem):
    wid = lax.axis_index("s") * 2 + lax.axis_index("c")
    lanes = lax.broadcasted_iota(jnp.int32, (LANES,), 0)
    zero = jnp.zeros((LANES,), jnp.int32)

    pltpu.sync_copy(route_hbm.at[pl.ds(wid * TOK_PT * 128, TOK_PT * 128)], rbuf)
    pltpu.sync_copy(hist_hbm, histbuf)

    totals = zero
    mybase = zero
    widv = jnp.full((LANES,), wid, jnp.int32)
    for wp in range(NTILES):
        row = histbuf[wp, pl.ds(0, LANES)].astype(jnp.int32)
        mybase = mybase + jnp.where(jnp.full((LANES,), wp, jnp.int32) < widv, row, zero)
        totals = totals + row
    tiles_e = (totals + (BM - 1)) // BM
    pad_off = BM * (plsc.cumsum(tiles_e) - tiles_e)
    table[...] = pad_off + mybase

    @pl.when(wid == 0)
    def _():
        cntbuf[...] = totals
        pltpu.sync_copy(cntbuf, cnt_hbm)

    carry = [jnp.int32(0)] * E
    for j in range(SLOT_PT // LANES):
        e_vec = _slot_experts(rbuf, j)
        slot = 16 * j + lanes
        tok = slot // TOPK
        wv = plsc.load_gather(rbuf, [tok * 128 + 2 + (slot % TOPK)])
        wbuf[pl.ds(16 * j, LANES)] = wv
        posv = zero
        for e in range(E):
            m = (e_vec == e)
            mi = m.astype(jnp.int32)
            incl = plsc.cumsum(mi)
            posv = jnp.where(m, carry[e] + (incl - mi), posv)
            carry[e] = carry[e] + jnp.sum(mi)
        base = plsc.load_gather(table, [e_vec])
        gpos = base + posv
        posbuf[pl.ds(16 * j, LANES)] = gpos
        evenm = (lanes % 2) == 0
        dest = (16 * j + lanes) // 2
        plsc.store_scatter(idx_a, [dest], gpos, mask=evenm)
        plsc.store_scatter(idx_b, [dest], gpos, mask=~evenm)

    pltpu.sync_copy(posbuf, pos_hbm.at[pl.ds(wid * SLOT_PT, SLOT_PT)])

    pltpu.sync_copy(x_hbm.at[pl.ds(wid * TOK_PT, TOK_PT)], xrows)
    pltpu.async_copy(xrows, xpad_hbm.at[idx_a], sem).wait()
    pltpu.async_copy(xrows, xpad_hbm.at[idx_b], sem).wait()
    pltpu.async_copy(wbuf, wpad_hbm.at[posbuf], sem).wait()


def _dispatch(route, hist, x):
    mesh = plsc.VectorSubcoreMesh(core_axis_name="c", subcore_axis_name="s")
    f = functools.partial(
        pl.kernel,
        mesh=mesh,
        compiler_params=pltpu.CompilerParams(needs_layout_passes=False),
        out_type=[
            jax.ShapeDtypeStruct((NPAD, H), jnp.float32),
            jax.ShapeDtypeStruct((T * TOPK,), jnp.int32),
            jax.ShapeDtypeStruct((LANES,), jnp.int32),
            jax.ShapeDtypeStruct((NPAD,), jnp.float32),
        ],
        scratch_types=[
            pltpu.VMEM((TOK_PT * 128,), jnp.float32),
            pltpu.VMEM((NTILES, 128), jnp.float32),
            pltpu.VMEM((LANES,), jnp.int32),
            pltpu.VMEM((SLOT_PT,), jnp.int32),
            pltpu.VMEM((TOK_PT,), jnp.int32),
            pltpu.VMEM((TOK_PT,), jnp.int32),
            pltpu.VMEM((TOK_PT, H), jnp.float32),
            pltpu.VMEM((LANES,), jnp.int32),
            pltpu.VMEM((SLOT_PT,), jnp.float32),
            pltpu.SemaphoreType.DMA,
        ],
    )
    return f(_dispatch_kernel)(route, hist, x)


def _ffn_body(bexp_ref, act_ref, x_ref, w1_ref, b1_ref, w2_ref, b2_ref, w_ref,
              o_ref):
    @pl.when(act_ref[pl.program_id(0)] != 0)
    def _():
        dot = functools.partial(
            lax.dot_general,
            dimension_numbers=(((1,), (0,)), ((), ())),
            precision=lax.Precision.DEFAULT,
            preferred_element_type=jnp.float32)
        xb = x_ref[...]
        h = jax.nn.gelu(dot(xb, w1_ref[0]) + b1_ref[0])
        y = dot(h, w2_ref[0])
        o_ref[...] = (y + b2_ref[0]) * w_ref[...]


def _grouped_ffn(x_pad, W1, b1, W2, b2, w_pad, bexp, act):
    grid_spec = pltpu.PrefetchScalarGridSpec(
        num_scalar_prefetch=2,
        grid=(NBLK,),
        in_specs=[
            pl.BlockSpec((BM, H), lambda g, be, ac: (g, 0)),
            pl.BlockSpec((1, H, F), lambda g, be, ac: (be[g], 0, 0)),
            pl.BlockSpec((1, 1, F), lambda g, be, ac: (be[g], 0, 0)),
            pl.BlockSpec((1, F, H), lambda g, be, ac: (be[g], 0, 0)),
            pl.BlockSpec((1, 1, H), lambda g, be, ac: (be[g], 0, 0)),
            pl.BlockSpec((BM, 1), lambda g, be, ac: (g, 0)),
        ],
        out_specs=pl.BlockSpec((BM, H), lambda g, be, ac: (g, 0)),
    )
    return pl.pallas_call(
        _ffn_body,
        grid_spec=grid_spec,
        out_shape=jax.ShapeDtypeStruct((NPAD, H), jnp.float32),
    )(bexp, act, x_pad, W1, b1.reshape(E, 1, F), W2, b2.reshape(E, 1, H),
      w_pad.reshape(NPAD, 1))


CHUNK = 32


def _combine_kernel(ys_hbm, pos_hbm, out_hbm, idx, rows, outb, sem):
    wid = lax.axis_index("s") * 2 + lax.axis_index("c")
    lanes = lax.broadcasted_iota(jnp.int32, (LANES,), 0)

    for c in range(TOK_PT // CHUNK):
        base_tok = wid * TOK_PT + c * CHUNK
        pltpu.sync_copy(pos_hbm.at[pl.ds(2 * base_tok, 2 * CHUNK)], idx)
        pltpu.async_copy(ys_hbm.at[idx], rows, sem).wait()

        def body(i, _):
            for j in range(H // LANES):
                a = rows[2 * i, pl.ds(j * LANES, LANES)]
                b = rows[2 * i + 1, pl.ds(j * LANES, LANES)]
                outb[i, pl.ds(j * LANES, LANES)] = a + b
            return 0

        lax.fori_loop(0, CHUNK, body, 0)
        pltpu.sync_copy(outb, out_hbm.at[pl.ds(base_tok, CHUNK)])


def _combine(ys_pad, pos):
    mesh = plsc.VectorSubcoreMesh(core_axis_name="c", subcore_axis_name="s")
    f = functools.partial(
        pl.kernel,
        mesh=mesh,
        compiler_params=pltpu.CompilerParams(needs_layout_passes=False),
        out_type=jax.ShapeDtypeStruct((T, H), jnp.float32),
        scratch_types=[
            pltpu.VMEM((2 * CHUNK,), jnp.int32),
            pltpu.VMEM((2 * CHUNK, H), jnp.float32),
            pltpu.VMEM((CHUNK, H), jnp.float32),
            pltpu.SemaphoreType.DMA,
        ],
    )
    return f(_combine_kernel)(ys_pad, pos)


def kernel(x, Wr, W1, b1, W2, b2):
    route, hist = _router(x, Wr)
    x_pad, pos, counts, w_pad = _dispatch(
        route.reshape(-1), hist.reshape(NTILES, 128), x)

    counts8 = counts[:E]
    tiles = (counts8 + BM - 1) // BM
    cum_tiles = jnp.cumsum(tiles)
    g_range = jnp.arange(NBLK, dtype=jnp.int32)
    bexp = jnp.minimum(
        jnp.searchsorted(cum_tiles, g_range, side="right"), E - 1
    ).astype(jnp.int32)
    act = (g_range < cum_tiles[-1]).astype(jnp.int32)

    ys_pad = _grouped_ffn(x_pad, W1, b1, W2, b2, w_pad, bexp, act)
    return _combine(ys_pad, pos)
